# baseline (device time: 29661 ns/iter reference)
import jax
import jax.numpy as jnp
from jax import lax
from jax.experimental import pallas as pl
from jax.experimental.pallas import tpu as pltpu

N_DEV = 8
B, Sq, Skv = 2, 256, 256
HQ_PER, Dh = 4, 64
D_MODEL = 512
HEAD_BLK = HQ_PER * Dh
ROWS = B * Sq
HALF_C = D_MODEL // 2
WINDOW = 128
SCALE = 0.125

_HALVES = [ROWS // 2, ROWS // 4, ROWS // 8]
_ROFFS = [0, ROWS // 2, 3 * ROWS // 4]


def kernel(x, Wq, K_ext, V_ext, Wo):
    def body(x_ref, wq_ref, k_ref, v_ref, wo_ref, out_ref,
             acc_ref, recv_ref, send_sems, recv_sems):
        p = lax.axis_index("i")
        bit0 = p & 1
        bit1 = (p >> 1) & 1
        bit2 = (p >> 2) & 1

        barrier_sem = pltpu.get_barrier_semaphore()
        for m in (1, 3, 4):
            pl.semaphore_signal(
                barrier_sem, inc=1,
                device_id=(p ^ m,), device_id_type=pl.DeviceIdType.MESH,
            )

        qi = lax.broadcasted_iota(jnp.int32, (Sq, Skv), 0)
        ki = lax.broadcasted_iota(jnp.int32, (Sq, Skv), 1)
        bias = jnp.where(jnp.abs(qi - ki) <= WINDOW, 0.0, -1e9)
        col0 = p * HEAD_BLK

        def compute_partial(bt):
            bf16 = jnp.bfloat16
            xb = x_ref[pl.ds(bt, 1)][0].astype(bf16)
            kb = k_ref[pl.ds(bt, 1)][0].astype(bf16)
            vb = v_ref[pl.ds(bt, 1)][0].astype(bf16)
            qb = jnp.dot(xb, wq_ref[:, pl.ds(col0, HEAD_BLK)].astype(bf16),
                         preferred_element_type=jnp.float32)
            ctx_parts = []
            for h in range(HQ_PER):
                qh = qb[:, h * Dh:(h + 1) * Dh].astype(bf16)
                s = jnp.dot(qh, kb[:, h, :].T,
                            preferred_element_type=jnp.float32)
                w = jnp.exp(s * SCALE + bias)
                denom = jnp.sum(w, axis=-1, keepdims=True)
                ctx_parts.append(
                    jnp.dot(w.astype(bf16), vb[:, h, :],
                            preferred_element_type=jnp.float32) / denom)
            ctx = jnp.concatenate(ctx_parts, axis=1)
            part = jnp.dot(ctx.astype(bf16),
                           wo_ref[pl.ds(col0, HEAD_BLK), :].astype(bf16),
                           preferred_element_type=jnp.float32)
            acc_ref[pl.ds(pl.multiple_of(bt * Sq, Sq), Sq), :] = (
                part.astype(jnp.bfloat16))

        bfA = {"c0": 0, "masks": [1, 3, 4], "sem0": 0,
               "f": [(bit0 ^ bit1) == 1, bit1 == 1, bit2 == 1],
               "lo": jnp.int32(0)}
        bfB = {"c0": HALF_C, "masks": [4, 1, 3], "sem0": 6,
               "f": [bit2 == 1, (bit0 ^ bit1) == 1, bit1 == 1],
               "lo": jnp.int32(0)}

        def rs_start(bf, s):
            half = _HALVES[s]
            send_lo = pl.multiple_of(
                bf["lo"] + jnp.where(bf["f"][s], 0, half), 64)
            bf["keep_lo"] = pl.multiple_of(
                bf["lo"] + jnp.where(bf["f"][s], half, 0), 64)
            d = pltpu.make_async_remote_copy(
                src_ref=acc_ref.at[pl.ds(send_lo, half),
                                   pl.ds(bf["c0"], HALF_C)],
                dst_ref=recv_ref.at[pl.ds(_ROFFS[s], half),
                                    pl.ds(bf["c0"], HALF_C)],
                send_sem=send_sems.at[bf["sem0"] + s],
                recv_sem=recv_sems.at[bf["sem0"] + s],
                device_id=(p ^ bf["masks"][s],),
                device_id_type=pl.DeviceIdType.MESH,
            )
            d.start()
            bf["pend"] = d

        def rs_finish(bf, s):
            half = _HALVES[s]
            bf["pend"].wait()
            kl = bf["keep_lo"]
            cur = acc_ref[pl.ds(kl, half), pl.ds(bf["c0"], HALF_C)]
            rv = recv_ref[_ROFFS[s]:_ROFFS[s] + half,
                          bf["c0"]:bf["c0"] + HALF_C]
            acc_ref[pl.ds(kl, half), pl.ds(bf["c0"], HALF_C)] = cur + rv
            bf["lo"] = kl

        def ag_start(bf, s):
            sz = _HALVES[2 - s]
            mi = 2 - s
            sem = bf["sem0"] + 3 + s
            lo = pl.multiple_of(bf["lo"], 64)
            partner_lo = pl.multiple_of(
                jnp.where(bf["f"][mi], lo - sz, lo + sz), 64)
            send = pltpu.make_async_remote_copy(
                src_ref=acc_ref.at[pl.ds(lo, sz), pl.ds(bf["c0"], HALF_C)],
                dst_ref=acc_ref.at[pl.ds(lo, sz), pl.ds(bf["c0"], HALF_C)],
                send_sem=send_sems.at[sem],
                recv_sem=recv_sems.at[sem],
                device_id=(p ^ bf["masks"][mi],),
                device_id_type=pl.DeviceIdType.MESH,
            )
            send.start()
            recv = pltpu.make_async_remote_copy(
                src_ref=acc_ref.at[pl.ds(partner_lo, sz),
                                   pl.ds(bf["c0"], HALF_C)],
                dst_ref=acc_ref.at[pl.ds(partner_lo, sz),
                                   pl.ds(bf["c0"], HALF_C)],
                send_sem=send_sems.at[sem],
                recv_sem=recv_sems.at[sem],
                device_id=(p ^ bf["masks"][mi],),
                device_id_type=pl.DeviceIdType.MESH,
            )
            bf["pend"] = (send, recv)
            bf["lo"] = jnp.minimum(lo, partner_lo)

        def ag_finish(bf):
            send, recv = bf["pend"]
            recv.wait_recv()
            send.wait_send()

        b_first = jnp.where(bfA["f"][0], 0, 1)
        compute_partial(b_first)
        pl.semaphore_wait(barrier_sem, 3)
        rs_start(bfA, 0)
        compute_partial(1 - b_first)
        rs_start(bfB, 0)
        rs_finish(bfA, 0)
        rs_start(bfA, 1)
        rs_finish(bfB, 0)
        rs_start(bfB, 1)
        rs_finish(bfA, 1)
        rs_start(bfA, 2)
        rs_finish(bfB, 1)
        rs_start(bfB, 2)
        rs_finish(bfA, 2)
        ag_start(bfA, 0)
        rs_finish(bfB, 2)
        ag_start(bfB, 0)
        ag_finish(bfA)
        ag_start(bfA, 1)
        ag_finish(bfB)
        ag_start(bfB, 1)
        ag_finish(bfA)
        ag_start(bfA, 2)
        ag_finish(bfB)
        ag_start(bfB, 2)
        ag_finish(bfA)
        ag_finish(bfB)
        out_ref[:, :] = acc_ref[:, :].astype(jnp.float32)

    out_shape = jax.ShapeDtypeStruct((ROWS, D_MODEL), jnp.float32)
    res = pl.pallas_call(
        body,
        out_shape=out_shape,
        in_specs=[pl.BlockSpec(memory_space=pltpu.VMEM)] * 5,
        out_specs=pl.BlockSpec(memory_space=pltpu.VMEM),
        scratch_shapes=[
            pltpu.VMEM((ROWS, D_MODEL), jnp.bfloat16),
            pltpu.VMEM((7 * ROWS // 8, D_MODEL), jnp.bfloat16),
            pltpu.SemaphoreType.DMA((12,)),
            pltpu.SemaphoreType.DMA((12,)),
        ],
        compiler_params=pltpu.CompilerParams(collective_id=0),
    )(x, Wq, K_ext, V_ext, Wo)
    return res.reshape(B, Sq, D_MODEL)


# device time: 29115 ns/iter; 1.0188x vs baseline; 1.0188x over previous
import jax
import jax.numpy as jnp
from jax import lax
from jax.experimental import pallas as pl
from jax.experimental.pallas import tpu as pltpu

N_DEV = 8
B, Sq, Skv = 2, 256, 256
HQ_PER, Dh = 4, 64
D_MODEL = 512
HEAD_BLK = HQ_PER * Dh
ROWS = B * Sq
HALF_R = ROWS // 2
HALF_C = D_MODEL // 2
WINDOW = 128
SCALE = 0.125


def kernel(x, Wq, K_ext, V_ext, Wo):
    def body(x_ref, wq_ref, k_ref, v_ref, wo_ref, out_ref,
             acc_ref, recv_ref, send_sems, recv_sems):
        p = lax.axis_index("i")
        f0 = ((p & 1) ^ ((p >> 1) & 1)) == 1

        barrier_sem = pltpu.get_barrier_semaphore()
        for m in (1, 3, 4):
            pl.semaphore_signal(
                barrier_sem, inc=1,
                device_id=(p ^ m,), device_id_type=pl.DeviceIdType.MESH,
            )

        qi = lax.broadcasted_iota(jnp.int32, (Sq, Skv), 0)
        ki = lax.broadcasted_iota(jnp.int32, (Sq, Skv), 1)
        bias = jnp.where(jnp.abs(qi - ki) <= WINDOW, 0.0, -1e9)
        col0 = p * HEAD_BLK

        def compute_partial(bt):
            xb = x_ref[pl.ds(bt, 1)][0]
            kb = k_ref[pl.ds(bt, 1)][0]
            vb = v_ref[pl.ds(bt, 1)][0]
            qb = jnp.dot(xb, wq_ref[:, pl.ds(col0, HEAD_BLK)],
                         preferred_element_type=jnp.float32)
            ctx_parts = []
            for h in range(HQ_PER):
                qh = qb[:, h * Dh:(h + 1) * Dh]
                s = jnp.dot(qh, kb[:, h, :].T,
                            preferred_element_type=jnp.float32)
                w = jnp.exp(s * SCALE + bias)
                denom = jnp.sum(w, axis=-1, keepdims=True)
                ctx_parts.append(
                    jnp.dot(w, vb[:, h, :],
                            preferred_element_type=jnp.float32) / denom)
            ctx = jnp.concatenate(ctx_parts, axis=1)
            part = jnp.dot(ctx, wo_ref[pl.ds(col0, HEAD_BLK), :],
                           preferred_element_type=jnp.float32)
            acc_ref[pl.ds(pl.multiple_of(bt * Sq, Sq), Sq), :] = (
                part.astype(jnp.bfloat16))

        b_first = jnp.where(f0, 0, 1)
        send_lo = pl.multiple_of(b_first * HALF_R, 256)
        keep_lo = pl.multiple_of((1 - b_first) * HALF_R, 256)

        def exchange(src_slc, dst_slc, sem, partner, recv_dst_slc=None):
            send = pltpu.make_async_remote_copy(
                src_ref=src_slc, dst_ref=dst_slc,
                send_sem=send_sems.at[sem], recv_sem=recv_sems.at[sem],
                device_id=(partner,), device_id_type=pl.DeviceIdType.MESH,
            )
            send.start()
            if recv_dst_slc is None:
                return send, send
            recv = pltpu.make_async_remote_copy(
                src_ref=recv_dst_slc, dst_ref=recv_dst_slc,
                send_sem=send_sems.at[sem], recv_sem=recv_sems.at[sem],
                device_id=(partner,), device_id_type=pl.DeviceIdType.MESH,
            )
            return send, recv

        compute_partial(b_first)
        pl.semaphore_wait(barrier_sem, 3)
        s0, _ = exchange(
            acc_ref.at[pl.ds(send_lo, HALF_R)],
            recv_ref.at[pl.ds(0, HALF_R)],
            0, p ^ 1,
        )
        compute_partial(1 - b_first)
        s0.wait()
        acc_ref[pl.ds(keep_lo, HALF_R), :] = (
            acc_ref[pl.ds(keep_lo, HALF_R), :]
            + recv_ref[pl.ds(0, HALF_R), :])

        def fe_start(c0, sem, partner, roff):
            s, _ = exchange(
                acc_ref.at[pl.ds(keep_lo, HALF_R), pl.ds(c0, HALF_C)],
                recv_ref.at[pl.ds(roff, HALF_R), pl.ds(c0, HALF_C)],
                sem, partner,
            )
            return s

        def fe_finish(c0, d, roff):
            d.wait()
            acc_ref[pl.ds(keep_lo, HALF_R), pl.ds(c0, HALF_C)] = (
                acc_ref[pl.ds(keep_lo, HALF_R), pl.ds(c0, HALF_C)]
                + recv_ref[pl.ds(roff, HALF_R), pl.ds(c0, HALF_C)])

        a1 = fe_start(0, 1, p ^ 3, HALF_R)
        b1 = fe_start(HALF_C, 3, p ^ 4, HALF_R)
        fe_finish(0, a1, HALF_R)
        a2 = fe_start(0, 2, p ^ 4, 2 * HALF_R)
        fe_finish(HALF_C, b1, HALF_R)
        b2 = fe_start(HALF_C, 4, p ^ 3, 2 * HALF_R)
        fe_finish(0, a2, 2 * HALF_R)
        fe_finish(HALF_C, b2, 2 * HALF_R)

        ag_send, ag_recv = exchange(
            acc_ref.at[pl.ds(keep_lo, HALF_R)],
            acc_ref.at[pl.ds(keep_lo, HALF_R)],
            5, p ^ 1,
            recv_dst_slc=acc_ref.at[pl.ds(send_lo, HALF_R)],
        )
        out_ref[pl.ds(keep_lo, HALF_R), :] = (
            acc_ref[pl.ds(keep_lo, HALF_R), :].astype(jnp.float32))
        ag_recv.wait_recv()
        ag_send.wait_send()
        out_ref[pl.ds(send_lo, HALF_R), :] = (
            acc_ref[pl.ds(send_lo, HALF_R), :].astype(jnp.float32))

    out_shape = jax.ShapeDtypeStruct((ROWS, D_MODEL), jnp.float32)
    res = pl.pallas_call(
        body,
        out_shape=out_shape,
        in_specs=[pl.BlockSpec(memory_space=pltpu.VMEM)] * 5,
        out_specs=pl.BlockSpec(memory_space=pltpu.VMEM),
        scratch_shapes=[
            pltpu.VMEM((ROWS, D_MODEL), jnp.bfloat16),
            pltpu.VMEM((3 * HALF_R, D_MODEL), jnp.bfloat16),
            pltpu.SemaphoreType.DMA((6,)),
            pltpu.SemaphoreType.DMA((6,)),
        ],
        compiler_params=pltpu.CompilerParams(collective_id=0),
    )(x, Wq, K_ext, V_ext, Wo)
    return res.reshape(B, Sq, D_MODEL)


# device time: 27960 ns/iter; 1.0608x vs baseline; 1.0413x over previous
import jax
import jax.numpy as jnp
from jax import lax
from jax.experimental import pallas as pl
from jax.experimental.pallas import tpu as pltpu

N_DEV = 8
B, Sq, Skv = 2, 256, 256
HQ_PER, Dh = 4, 64
D_MODEL = 512
HEAD_BLK = HQ_PER * Dh
ROWS = B * Sq
HALF_R = ROWS // 2
HALF_C = D_MODEL // 2
WINDOW = 128
SCALE = 0.125


def kernel(x, Wq, K_ext, V_ext, Wo):
    def body(x_ref, wq_ref, k_ref, v_ref, wo_ref, out_ref,
             acc_ref, recv_ref, ctx_ref, ctx_rcv_ref,
             send_sems, recv_sems):
        p = lax.axis_index("i")
        f0 = ((p & 1) ^ ((p >> 1) & 1)) == 1

        barrier_sem = pltpu.get_barrier_semaphore()
        for m in (1, 3, 4):
            pl.semaphore_signal(
                barrier_sem, inc=1,
                device_id=(p ^ m,), device_id_type=pl.DeviceIdType.MESH,
            )

        qi = lax.broadcasted_iota(jnp.int32, (Sq, Skv), 0)
        ki = lax.broadcasted_iota(jnp.int32, (Sq, Skv), 1)
        bias = jnp.where(jnp.abs(qi - ki) <= WINDOW, 0.0, -1e9)
        col0 = p * HEAD_BLK

        def compute_attn(bt):
            xb = x_ref[pl.ds(bt, 1)][0]
            kb = k_ref[pl.ds(bt, 1)][0]
            vb = v_ref[pl.ds(bt, 1)][0]
            qb = jnp.dot(xb, wq_ref[:, pl.ds(col0, HEAD_BLK)],
                         preferred_element_type=jnp.float32)
            ctx_parts = []
            for h in range(HQ_PER):
                qh = qb[:, h * Dh:(h + 1) * Dh]
                s = jnp.dot(qh, kb[:, h, :].T,
                            preferred_element_type=jnp.float32)
                w = jnp.exp(s * SCALE + bias)
                denom = jnp.sum(w, axis=-1, keepdims=True)
                ctx_parts.append(
                    jnp.dot(w, vb[:, h, :],
                            preferred_element_type=jnp.float32) / denom)
            return jnp.concatenate(ctx_parts, axis=1)

        b_first = jnp.where(f0, 0, 1)
        send_lo = pl.multiple_of(b_first * HALF_R, 256)
        keep_lo = pl.multiple_of((1 - b_first) * HALF_R, 256)

        def exchange(src_slc, dst_slc, sem, partner, recv_dst_slc=None):
            send = pltpu.make_async_remote_copy(
                src_ref=src_slc, dst_ref=dst_slc,
                send_sem=send_sems.at[sem], recv_sem=recv_sems.at[sem],
                device_id=(partner,), device_id_type=pl.DeviceIdType.MESH,
            )
            send.start()
            if recv_dst_slc is None:
                return send, send
            recv = pltpu.make_async_remote_copy(
                src_ref=recv_dst_slc, dst_ref=recv_dst_slc,
                send_sem=send_sems.at[sem], recv_sem=recv_sems.at[sem],
                device_id=(partner,), device_id_type=pl.DeviceIdType.MESH,
            )
            return send, recv

        ctx_ref[:, :] = compute_attn(b_first).astype(jnp.bfloat16)
        pl.semaphore_wait(barrier_sem, 3)
        s0, _ = exchange(ctx_ref, ctx_rcv_ref, 0, p ^ 1)
        ctx_keep = compute_attn(1 - b_first)
        s0.wait()
        colq = (p ^ 1) * HEAD_BLK
        part = (
            jnp.dot(ctx_keep, wo_ref[pl.ds(col0, HEAD_BLK), :],
                    preferred_element_type=jnp.float32)
            + jnp.dot(ctx_rcv_ref[:, :],
                      wo_ref[pl.ds(colq, HEAD_BLK), :].astype(jnp.bfloat16),
                      preferred_element_type=jnp.float32))
        acc_ref[pl.ds(keep_lo, HALF_R), :] = part.astype(jnp.bfloat16)

        def fe_start(c0, sem, partner, roff):
            s, _ = exchange(
                acc_ref.at[pl.ds(keep_lo, HALF_R), pl.ds(c0, HALF_C)],
                recv_ref.at[pl.ds(roff, HALF_R), pl.ds(c0, HALF_C)],
                sem, partner,
            )
            return s

        def fe_finish(c0, d, roff):
            d.wait()
            acc_ref[pl.ds(keep_lo, HALF_R), pl.ds(c0, HALF_C)] = (
                acc_ref[pl.ds(keep_lo, HALF_R), pl.ds(c0, HALF_C)]
                + recv_ref[pl.ds(roff, HALF_R), pl.ds(c0, HALF_C)])

        a1 = fe_start(0, 1, p ^ 3, 0)
        b1 = fe_start(HALF_C, 3, p ^ 4, 0)
        fe_finish(0, a1, 0)
        a2 = fe_start(0, 2, p ^ 4, HALF_R)
        fe_finish(HALF_C, b1, 0)
        b2 = fe_start(HALF_C, 4, p ^ 3, HALF_R)
        fe_finish(0, a2, HALF_R)
        fe_finish(HALF_C, b2, HALF_R)

        ag_send, ag_recv = exchange(
            acc_ref.at[pl.ds(keep_lo, HALF_R)],
            acc_ref.at[pl.ds(keep_lo, HALF_R)],
            5, p ^ 1,
            recv_dst_slc=acc_ref.at[pl.ds(send_lo, HALF_R)],
        )
        out_ref[pl.ds(keep_lo, HALF_R), :] = (
            acc_ref[pl.ds(keep_lo, HALF_R), :].astype(jnp.float32))
        ag_recv.wait_recv()
        ag_send.wait_send()
        out_ref[pl.ds(send_lo, HALF_R), :] = (
            acc_ref[pl.ds(send_lo, HALF_R), :].astype(jnp.float32))

    out_shape = jax.ShapeDtypeStruct((ROWS, D_MODEL), jnp.float32)
    res = pl.pallas_call(
        body,
        out_shape=out_shape,
        in_specs=[pl.BlockSpec(memory_space=pltpu.VMEM)] * 5,
        out_specs=pl.BlockSpec(memory_space=pltpu.VMEM),
        scratch_shapes=[
            pltpu.VMEM((ROWS, D_MODEL), jnp.bfloat16),
            pltpu.VMEM((2 * HALF_R, D_MODEL), jnp.bfloat16),
            pltpu.VMEM((Sq, HEAD_BLK), jnp.bfloat16),
            pltpu.VMEM((Sq, HEAD_BLK), jnp.bfloat16),
            pltpu.SemaphoreType.DMA((6,)),
            pltpu.SemaphoreType.DMA((6,)),
        ],
        compiler_params=pltpu.CompilerParams(collective_id=0),
    )(x, Wq, K_ext, V_ext, Wo)
    return res.reshape(B, Sq, D_MODEL)


# device time: 27835 ns/iter; 1.0656x vs baseline; 1.0045x over previous
import jax
import jax.numpy as jnp
from jax import lax
from jax.experimental import pallas as pl
from jax.experimental.pallas import tpu as pltpu

N_DEV = 8
B, Sq, Skv = 2, 256, 256
HQ_PER, Dh = 4, 64
D_MODEL = 512
HEAD_BLK = HQ_PER * Dh
ROWS = B * Sq
HALF_R = ROWS // 2
HALF_C = D_MODEL // 2
WINDOW = 128
SCALE = 0.125


def kernel(x, Wq, K_ext, V_ext, Wo):
    def body(x_ref, wq_ref, k_ref, v_ref, wo_ref, out_ref,
             acc_ref, recv_ref, ctx_ref, ctx_rcv_ref,
             send_sems, recv_sems):
        p = lax.axis_index("i")
        f0 = ((p & 1) ^ ((p >> 1) & 1)) == 1

        barrier_sem = pltpu.get_barrier_semaphore()
        for m in (1, 3, 4):
            pl.semaphore_signal(
                barrier_sem, inc=1,
                device_id=(p ^ m,), device_id_type=pl.DeviceIdType.MESH,
            )

        qi = lax.broadcasted_iota(jnp.int32, (Sq, Skv), 0)
        ki = lax.broadcasted_iota(jnp.int32, (Sq, Skv), 1)
        bias = jnp.where(jnp.abs(qi - ki) <= WINDOW, 0.0, -1e9)
        col0 = p * HEAD_BLK

        def compute_attn(bt):
            xb = x_ref[pl.ds(bt, 1)][0]
            kb = k_ref[pl.ds(bt, 1)][0]
            vb = v_ref[pl.ds(bt, 1)][0]
            qb = jnp.dot(xb.astype(jnp.bfloat16),
                         wq_ref[:, pl.ds(col0, HEAD_BLK)].astype(jnp.bfloat16),
                         preferred_element_type=jnp.float32)
            ctx_parts = []
            for h in range(HQ_PER):
                qh = qb[:, h * Dh:(h + 1) * Dh]
                s = jnp.dot(qh, kb[:, h, :].T,
                            preferred_element_type=jnp.float32)
                w = jnp.exp(s * SCALE + bias)
                denom = jnp.sum(w, axis=-1, keepdims=True)
                ctx_parts.append(
                    jnp.dot(w, vb[:, h, :],
                            preferred_element_type=jnp.float32) / denom)
            return jnp.concatenate(ctx_parts, axis=1)

        b_first = jnp.where(f0, 0, 1)
        send_lo = pl.multiple_of(b_first * HALF_R, 256)
        keep_lo = pl.multiple_of((1 - b_first) * HALF_R, 256)

        def exchange(src_slc, dst_slc, sem, partner, recv_dst_slc=None):
            send = pltpu.make_async_remote_copy(
                src_ref=src_slc, dst_ref=dst_slc,
                send_sem=send_sems.at[sem], recv_sem=recv_sems.at[sem],
                device_id=(partner,), device_id_type=pl.DeviceIdType.MESH,
            )
            send.start()
            if recv_dst_slc is None:
                return send, send
            recv = pltpu.make_async_remote_copy(
                src_ref=recv_dst_slc, dst_ref=recv_dst_slc,
                send_sem=send_sems.at[sem], recv_sem=recv_sems.at[sem],
                device_id=(partner,), device_id_type=pl.DeviceIdType.MESH,
            )
            return send, recv

        ctx_ref[:, :] = compute_attn(b_first).astype(jnp.bfloat16)
        pl.semaphore_wait(barrier_sem, 3)
        s0, _ = exchange(ctx_ref, ctx_rcv_ref, 0, p ^ 1)
        ctx_keep = compute_attn(1 - b_first)
        s0.wait()
        colq = (p ^ 1) * HEAD_BLK
        part = (
            jnp.dot(ctx_keep.astype(jnp.bfloat16),
                    wo_ref[pl.ds(col0, HEAD_BLK), :].astype(jnp.bfloat16),
                    preferred_element_type=jnp.float32)
            + jnp.dot(ctx_rcv_ref[:, :],
                      wo_ref[pl.ds(colq, HEAD_BLK), :].astype(jnp.bfloat16),
                      preferred_element_type=jnp.float32))
        acc_ref[pl.ds(keep_lo, HALF_R), :] = part.astype(jnp.bfloat16)

        def fe_start(c0, sem, partner, roff):
            s, _ = exchange(
                acc_ref.at[pl.ds(keep_lo, HALF_R), pl.ds(c0, HALF_C)],
                recv_ref.at[pl.ds(roff, HALF_R), pl.ds(c0, HALF_C)],
                sem, partner,
            )
            return s

        def fe_finish(c0, d, roff):
            d.wait()
            acc_ref[pl.ds(keep_lo, HALF_R), pl.ds(c0, HALF_C)] = (
                acc_ref[pl.ds(keep_lo, HALF_R), pl.ds(c0, HALF_C)]
                + recv_ref[pl.ds(roff, HALF_R), pl.ds(c0, HALF_C)])

        a1 = fe_start(0, 1, p ^ 3, 0)
        b1 = fe_start(HALF_C, 3, p ^ 4, 0)
        fe_finish(0, a1, 0)
        a2 = fe_start(0, 2, p ^ 4, HALF_R)
        fe_finish(HALF_C, b1, 0)
        b2 = fe_start(HALF_C, 4, p ^ 3, HALF_R)
        fe_finish(0, a2, HALF_R)
        fe_finish(HALF_C, b2, HALF_R)

        ag_send, ag_recv = exchange(
            acc_ref.at[pl.ds(keep_lo, HALF_R)],
            acc_ref.at[pl.ds(keep_lo, HALF_R)],
            5, p ^ 1,
            recv_dst_slc=acc_ref.at[pl.ds(send_lo, HALF_R)],
        )
        out_ref[pl.ds(keep_lo, HALF_R), :] = (
            acc_ref[pl.ds(keep_lo, HALF_R), :].astype(jnp.float32))
        ag_recv.wait_recv()
        ag_send.wait_send()
        out_ref[pl.ds(send_lo, HALF_R), :] = (
            acc_ref[pl.ds(send_lo, HALF_R), :].astype(jnp.float32))

    out_shape = jax.ShapeDtypeStruct((ROWS, D_MODEL), jnp.float32)
    res = pl.pallas_call(
        body,
        out_shape=out_shape,
        in_specs=[pl.BlockSpec(memory_space=pltpu.VMEM)] * 5,
        out_specs=pl.BlockSpec(memory_space=pltpu.VMEM),
        scratch_shapes=[
            pltpu.VMEM((ROWS, D_MODEL), jnp.bfloat16),
            pltpu.VMEM((2 * HALF_R, D_MODEL), jnp.bfloat16),
            pltpu.VMEM((Sq, HEAD_BLK), jnp.bfloat16),
            pltpu.VMEM((Sq, HEAD_BLK), jnp.bfloat16),
            pltpu.SemaphoreType.DMA((6,)),
            pltpu.SemaphoreType.DMA((6,)),
        ],
        compiler_params=pltpu.CompilerParams(collective_id=0),
    )(x, Wq, K_ext, V_ext, Wo)
    return res.reshape(B, Sq, D_MODEL)


# device time: 26458 ns/iter; 1.1211x vs baseline; 1.0520x over previous
import jax
import jax.numpy as jnp
from jax import lax
from jax.experimental import pallas as pl
from jax.experimental.pallas import tpu as pltpu

N_DEV = 8
B, Sq, Skv = 2, 256, 256
HQ_PER, Dh = 4, 64
D_MODEL = 512
HEAD_BLK = HQ_PER * Dh
ROWS = B * Sq
HALF_R = ROWS // 2
HALF_C = D_MODEL // 2
WINDOW = 128
SCALE = 0.125


def kernel(x, Wq, K_ext, V_ext, Wo):
    def body(x_ref, wq_ref, k_ref, v_ref, wo_ref, out_ref,
             acc_ref, recv_ref, ctx_ref, ctx_rcv_ref,
             send_sems, recv_sems):
        p = lax.axis_index("i")
        f0 = ((p & 1) ^ ((p >> 1) & 1)) == 1

        barrier_sem = pltpu.get_barrier_semaphore()
        for m in (1, 3, 4):
            pl.semaphore_signal(
                barrier_sem, inc=1,
                device_id=(p ^ m,), device_id_type=pl.DeviceIdType.MESH,
            )

        qi = lax.broadcasted_iota(jnp.int32, (Sq, Skv), 0)
        ki = lax.broadcasted_iota(jnp.int32, (Sq, Skv), 1)
        bias = jnp.where(jnp.abs(qi - ki) <= WINDOW, 0.0, -1e9)
        col0 = p * HEAD_BLK

        def compute_attn(bt):
            xb = x_ref[pl.ds(bt, 1)][0]
            kb = k_ref[pl.ds(bt, 1)][0]
            vb = v_ref[pl.ds(bt, 1)][0]
            qb = jnp.dot(xb.astype(jnp.bfloat16),
                         wq_ref[:, pl.ds(col0, HEAD_BLK)].astype(jnp.bfloat16),
                         preferred_element_type=jnp.float32)
            ctx_parts = []
            for h in range(HQ_PER):
                qh = qb[:, h * Dh:(h + 1) * Dh]
                s = jnp.dot(qh, kb[:, h, :].T,
                            preferred_element_type=jnp.float32)
                w = jnp.exp(s * SCALE + bias)
                denom = jnp.sum(w, axis=-1, keepdims=True)
                ctx_parts.append(
                    jnp.dot(w, vb[:, h, :],
                            preferred_element_type=jnp.float32) / denom)
            return jnp.concatenate(ctx_parts, axis=1)

        b_first = jnp.where(f0, 0, 1)
        send_lo = pl.multiple_of(b_first * HALF_R, 256)
        keep_lo = pl.multiple_of((1 - b_first) * HALF_R, 256)

        def exchange(src_slc, dst_slc, sem, partner, recv_dst_slc=None):
            send = pltpu.make_async_remote_copy(
                src_ref=src_slc, dst_ref=dst_slc,
                send_sem=send_sems.at[sem], recv_sem=recv_sems.at[sem],
                device_id=(partner,), device_id_type=pl.DeviceIdType.MESH,
            )
            send.start()
            if recv_dst_slc is None:
                return send, send
            recv = pltpu.make_async_remote_copy(
                src_ref=recv_dst_slc, dst_ref=recv_dst_slc,
                send_sem=send_sems.at[sem], recv_sem=recv_sems.at[sem],
                device_id=(partner,), device_id_type=pl.DeviceIdType.MESH,
            )
            return send, recv

        ctx_ref[:, :] = compute_attn(b_first).astype(jnp.bfloat16)
        pl.semaphore_wait(barrier_sem, 3)
        s0, _ = exchange(ctx_ref, ctx_rcv_ref, 0, p ^ 1)
        ctx_keep = compute_attn(1 - b_first)
        s0.wait()
        colq = (p ^ 1) * HEAD_BLK
        wo_p = wo_ref[pl.ds(col0, HEAD_BLK), :].astype(jnp.bfloat16)
        wo_q = wo_ref[pl.ds(colq, HEAD_BLK), :].astype(jnp.bfloat16)

        CH = HALF_R // 2

        def klo(ci):
            return pl.multiple_of(keep_lo + ci * CH, 64)

        def fe_start(ci, c0, sem, partner, roff):
            s, _ = exchange(
                acc_ref.at[pl.ds(klo(ci), CH), pl.ds(c0, HALF_C)],
                recv_ref.at[pl.ds(roff + ci * CH, CH), pl.ds(c0, HALF_C)],
                sem, partner,
            )
            return s

        def fe_finish(ci, c0, d, roff):
            d.wait()
            acc_ref[pl.ds(klo(ci), CH), pl.ds(c0, HALF_C)] = (
                acc_ref[pl.ds(klo(ci), CH), pl.ds(c0, HALF_C)]
                + recv_ref[pl.ds(roff + ci * CH, CH), pl.ds(c0, HALF_C)])

        a1, b1, a2, b2, ag = {}, {}, {}, {}, {}
        for ci in range(2):
            part = (
                jnp.dot(ctx_keep[ci * CH:(ci + 1) * CH, :].astype(jnp.bfloat16),
                        wo_p, preferred_element_type=jnp.float32)
                + jnp.dot(ctx_rcv_ref[pl.ds(ci * CH, CH), :], wo_q,
                          preferred_element_type=jnp.float32))
            acc_ref[pl.ds(klo(ci), CH), :] = part.astype(jnp.bfloat16)
            a1[ci] = fe_start(ci, 0, 1 + ci, p ^ 3, 0)
            b1[ci] = fe_start(ci, HALF_C, 3 + ci, p ^ 4, 0)
        for ci in range(2):
            fe_finish(ci, 0, a1[ci], 0)
            a2[ci] = fe_start(ci, 0, 5 + ci, p ^ 4, HALF_R)
            fe_finish(ci, HALF_C, b1[ci], 0)
            b2[ci] = fe_start(ci, HALF_C, 7 + ci, p ^ 3, HALF_R)
        for ci in range(2):
            fe_finish(ci, 0, a2[ci], HALF_R)
            fe_finish(ci, HALF_C, b2[ci], HALF_R)
            slo = pl.multiple_of(send_lo + ci * CH, 64)
            ag[ci] = exchange(
                acc_ref.at[pl.ds(klo(ci), CH)],
                acc_ref.at[pl.ds(klo(ci), CH)],
                9 + ci, p ^ 1,
                recv_dst_slc=acc_ref.at[pl.ds(slo, CH)],
            )
            out_ref[pl.ds(klo(ci), CH), :] = (
                acc_ref[pl.ds(klo(ci), CH), :].astype(jnp.float32))
        for ci in range(2):
            ag_send, ag_recv = ag[ci]
            ag_recv.wait_recv()
            ag_send.wait_send()
            slo = pl.multiple_of(send_lo + ci * CH, 64)
            out_ref[pl.ds(slo, CH), :] = (
                acc_ref[pl.ds(slo, CH), :].astype(jnp.float32))

    out_shape = jax.ShapeDtypeStruct((ROWS, D_MODEL), jnp.float32)
    res = pl.pallas_call(
        body,
        out_shape=out_shape,
        in_specs=[pl.BlockSpec(memory_space=pltpu.VMEM)] * 5,
        out_specs=pl.BlockSpec(memory_space=pltpu.VMEM),
        scratch_shapes=[
            pltpu.VMEM((ROWS, D_MODEL), jnp.bfloat16),
            pltpu.VMEM((2 * HALF_R, D_MODEL), jnp.bfloat16),
            pltpu.VMEM((Sq, HEAD_BLK), jnp.bfloat16),
            pltpu.VMEM((Sq, HEAD_BLK), jnp.bfloat16),
            pltpu.SemaphoreType.DMA((11,)),
            pltpu.SemaphoreType.DMA((11,)),
        ],
        compiler_params=pltpu.CompilerParams(collective_id=0),
    )(x, Wq, K_ext, V_ext, Wo)
    return res.reshape(B, Sq, D_MODEL)


# device time: 26399 ns/iter; 1.1236x vs baseline; 1.0022x over previous
import jax
import jax.numpy as jnp
from jax import lax
from jax.experimental import pallas as pl
from jax.experimental.pallas import tpu as pltpu

N_DEV = 8
B, Sq, Skv = 2, 256, 256
HQ_PER, Dh = 4, 64
D_MODEL = 512
HEAD_BLK = HQ_PER * Dh
ROWS = B * Sq
HALF_R = ROWS // 2
HALF_C = D_MODEL // 2
WINDOW = 128
SCALE = 0.125


def kernel(x, Wq, K_ext, V_ext, Wo):
    def body(x_ref, wq_ref, k_ref, v_ref, wo_ref, out_ref,
             acc_ref, recv_ref, ctx_ref, ctx_rcv_ref,
             send_sems, recv_sems):
        p = lax.axis_index("i")
        f0 = ((p & 1) ^ ((p >> 1) & 1)) == 1

        barrier_sem = pltpu.get_barrier_semaphore()
        for m in (1, 3, 4):
            pl.semaphore_signal(
                barrier_sem, inc=1,
                device_id=(p ^ m,), device_id_type=pl.DeviceIdType.MESH,
            )

        qi = lax.broadcasted_iota(jnp.int32, (Sq, Skv), 0)
        ki = lax.broadcasted_iota(jnp.int32, (Sq, Skv), 1)
        bias = jnp.where(jnp.abs(qi - ki) <= WINDOW, 0.0, -1e9)
        col0 = p * HEAD_BLK

        CH = HALF_R // 2

        def compute_attn(bt, r0):
            xb = x_ref[pl.ds(bt, 1)][0][r0:r0 + CH, :]
            kb = k_ref[pl.ds(bt, 1)][0]
            vb = v_ref[pl.ds(bt, 1)][0]
            qb = jnp.dot(xb.astype(jnp.bfloat16),
                         wq_ref[:, pl.ds(col0, HEAD_BLK)].astype(jnp.bfloat16),
                         preferred_element_type=jnp.float32)
            ctx_parts = []
            for h in range(HQ_PER):
                qh = qb[:, h * Dh:(h + 1) * Dh]
                s = jnp.dot(qh, kb[:, h, :].T,
                            preferred_element_type=jnp.float32)
                w = jnp.exp(s * SCALE + bias[r0:r0 + CH, :])
                denom = jnp.sum(w, axis=-1, keepdims=True)
                ctx_parts.append(
                    jnp.dot(w, vb[:, h, :],
                            preferred_element_type=jnp.float32) / denom)
            return jnp.concatenate(ctx_parts, axis=1)

        b_first = jnp.where(f0, 0, 1)
        send_lo = pl.multiple_of(b_first * HALF_R, 256)
        keep_lo = pl.multiple_of((1 - b_first) * HALF_R, 256)

        def exchange(src_slc, dst_slc, sem, partner, recv_dst_slc=None):
            send = pltpu.make_async_remote_copy(
                src_ref=src_slc, dst_ref=dst_slc,
                send_sem=send_sems.at[sem], recv_sem=recv_sems.at[sem],
                device_id=(partner,), device_id_type=pl.DeviceIdType.MESH,
            )
            send.start()
            if recv_dst_slc is None:
                return send, send
            recv = pltpu.make_async_remote_copy(
                src_ref=recv_dst_slc, dst_ref=recv_dst_slc,
                send_sem=send_sems.at[sem], recv_sem=recv_sems.at[sem],
                device_id=(partner,), device_id_type=pl.DeviceIdType.MESH,
            )
            return send, recv

        ctx_ref[0:CH, :] = compute_attn(b_first, 0).astype(jnp.bfloat16)
        pl.semaphore_wait(barrier_sem, 3)
        s0a, _ = exchange(ctx_ref.at[pl.ds(0, CH)],
                          ctx_rcv_ref.at[pl.ds(0, CH)], 0, p ^ 1)
        ctx_ref[CH:2 * CH, :] = compute_attn(b_first, CH).astype(jnp.bfloat16)
        s0b, _ = exchange(ctx_ref.at[pl.ds(CH, CH)],
                          ctx_rcv_ref.at[pl.ds(CH, CH)], 11, p ^ 1)
        colq = (p ^ 1) * HEAD_BLK
        wo_p = wo_ref[pl.ds(col0, HEAD_BLK), :].astype(jnp.bfloat16)
        wo_q = wo_ref[pl.ds(colq, HEAD_BLK), :].astype(jnp.bfloat16)
        ck = {0: compute_attn(1 - b_first, 0)}
        s0 = {0: s0a, 1: s0b}

        def klo(ci):
            return pl.multiple_of(keep_lo + ci * CH, 64)

        def fe_start(ci, c0, sem, partner, roff):
            s, _ = exchange(
                acc_ref.at[pl.ds(klo(ci), CH), pl.ds(c0, HALF_C)],
                recv_ref.at[pl.ds(roff + ci * CH, CH), pl.ds(c0, HALF_C)],
                sem, partner,
            )
            return s

        def fe_finish(ci, c0, d, roff):
            d.wait()
            acc_ref[pl.ds(klo(ci), CH), pl.ds(c0, HALF_C)] = (
                acc_ref[pl.ds(klo(ci), CH), pl.ds(c0, HALF_C)]
                + recv_ref[pl.ds(roff + ci * CH, CH), pl.ds(c0, HALF_C)])

        a1, b1, a2, b2, ag = {}, {}, {}, {}, {}
        for ci in range(2):
            s0[ci].wait()
            part = (
                jnp.dot(ck[ci].astype(jnp.bfloat16),
                        wo_p, preferred_element_type=jnp.float32)
                + jnp.dot(ctx_rcv_ref[pl.ds(ci * CH, CH), :], wo_q,
                          preferred_element_type=jnp.float32))
            acc_ref[pl.ds(klo(ci), CH), :] = part.astype(jnp.bfloat16)
            a1[ci] = fe_start(ci, 0, 1 + ci, p ^ 3, 0)
            b1[ci] = fe_start(ci, HALF_C, 3 + ci, p ^ 4, 0)
            if ci == 0:
                ck[1] = compute_attn(1 - b_first, CH)
        for ci in range(2):
            fe_finish(ci, 0, a1[ci], 0)
            a2[ci] = fe_start(ci, 0, 5 + ci, p ^ 4, HALF_R)
            fe_finish(ci, HALF_C, b1[ci], 0)
            b2[ci] = fe_start(ci, HALF_C, 7 + ci, p ^ 3, HALF_R)
        for ci in range(2):
            fe_finish(ci, 0, a2[ci], HALF_R)
            fe_finish(ci, HALF_C, b2[ci], HALF_R)
            slo = pl.multiple_of(send_lo + ci * CH, 64)
            ag[ci] = exchange(
                acc_ref.at[pl.ds(klo(ci), CH)],
                acc_ref.at[pl.ds(klo(ci), CH)],
                9 + ci, p ^ 1,
                recv_dst_slc=acc_ref.at[pl.ds(slo, CH)],
            )
            out_ref[pl.ds(klo(ci), CH), :] = (
                acc_ref[pl.ds(klo(ci), CH), :].astype(jnp.float32))
        for ci in range(2):
            ag_send, ag_recv = ag[ci]
            ag_recv.wait_recv()
            ag_send.wait_send()
            slo = pl.multiple_of(send_lo + ci * CH, 64)
            out_ref[pl.ds(slo, CH), :] = (
                acc_ref[pl.ds(slo, CH), :].astype(jnp.float32))

    out_shape = jax.ShapeDtypeStruct((ROWS, D_MODEL), jnp.float32)
    res = pl.pallas_call(
        body,
        out_shape=out_shape,
        in_specs=[pl.BlockSpec(memory_space=pltpu.VMEM)] * 5,
        out_specs=pl.BlockSpec(memory_space=pltpu.VMEM),
        scratch_shapes=[
            pltpu.VMEM((ROWS, D_MODEL), jnp.bfloat16),
            pltpu.VMEM((2 * HALF_R, D_MODEL), jnp.bfloat16),
            pltpu.VMEM((Sq, HEAD_BLK), jnp.bfloat16),
            pltpu.VMEM((Sq, HEAD_BLK), jnp.bfloat16),
            pltpu.SemaphoreType.DMA((12,)),
            pltpu.SemaphoreType.DMA((12,)),
        ],
        compiler_params=pltpu.CompilerParams(collective_id=0),
    )(x, Wq, K_ext, V_ext, Wo)
    return res.reshape(B, Sq, D_MODEL)


# device time: 26395 ns/iter; 1.1237x vs baseline; 1.0002x over previous
import jax
import jax.numpy as jnp
from jax import lax
from jax.experimental import pallas as pl
from jax.experimental.pallas import tpu as pltpu

N_DEV = 8
B, Sq, Skv = 2, 256, 256
HQ_PER, Dh = 4, 64
D_MODEL = 512
HEAD_BLK = HQ_PER * Dh
ROWS = B * Sq
HALF_R = ROWS // 2
HALF_C = D_MODEL // 2
WINDOW = 128
SCALE = 0.125


def kernel(x, Wq, K_ext, V_ext, Wo):
    def body(x_ref, wq_ref, k_ref, v_ref, wo_ref, out_ref,
             acc_ref, recv_ref, ctx_ref, ctx_rcv_ref,
             send_sems, recv_sems, yz_sem):
        p = lax.axis_index("i")
        f0 = ((p & 1) ^ ((p >> 1) & 1)) == 1

        barrier_sem = pltpu.get_barrier_semaphore()
        pl.semaphore_signal(
            barrier_sem, inc=1,
            device_id=(p ^ 1,), device_id_type=pl.DeviceIdType.MESH,
        )
        for m in (3, 4):
            pl.semaphore_signal(
                yz_sem, inc=1,
                device_id=(p ^ m,), device_id_type=pl.DeviceIdType.MESH,
            )

        qi = lax.broadcasted_iota(jnp.int32, (Sq, Skv), 0)
        ki = lax.broadcasted_iota(jnp.int32, (Sq, Skv), 1)
        bias = jnp.where(jnp.abs(qi - ki) <= WINDOW, 0.0, -1e9)
        col0 = p * HEAD_BLK

        CH = HALF_R // 2

        def compute_attn(bt, r0):
            xb = x_ref[pl.ds(bt, 1)][0][r0:r0 + CH, :]
            kb = k_ref[pl.ds(bt, 1)][0]
            vb = v_ref[pl.ds(bt, 1)][0]
            qb = jnp.dot(xb.astype(jnp.bfloat16),
                         wq_ref[:, pl.ds(col0, HEAD_BLK)].astype(jnp.bfloat16),
                         preferred_element_type=jnp.float32)
            ctx_parts = []
            for h in range(HQ_PER):
                qh = qb[:, h * Dh:(h + 1) * Dh]
                s = jnp.dot(qh, kb[:, h, :].T,
                            preferred_element_type=jnp.float32)
                w = jnp.exp(s * SCALE + bias[r0:r0 + CH, :])
                denom = jnp.sum(w, axis=-1, keepdims=True)
                ctx_parts.append(
                    jnp.dot(w, vb[:, h, :],
                            preferred_element_type=jnp.float32) / denom)
            return jnp.concatenate(ctx_parts, axis=1)

        b_first = jnp.where(f0, 0, 1)
        send_lo = pl.multiple_of(b_first * HALF_R, 256)
        keep_lo = pl.multiple_of((1 - b_first) * HALF_R, 256)

        def exchange(src_slc, dst_slc, sem, partner, recv_dst_slc=None):
            send = pltpu.make_async_remote_copy(
                src_ref=src_slc, dst_ref=dst_slc,
                send_sem=send_sems.at[sem], recv_sem=recv_sems.at[sem],
                device_id=(partner,), device_id_type=pl.DeviceIdType.MESH,
            )
            send.start()
            if recv_dst_slc is None:
                return send, send
            recv = pltpu.make_async_remote_copy(
                src_ref=recv_dst_slc, dst_ref=recv_dst_slc,
                send_sem=send_sems.at[sem], recv_sem=recv_sems.at[sem],
                device_id=(partner,), device_id_type=pl.DeviceIdType.MESH,
            )
            return send, recv

        ctx_ref[0:CH, :] = compute_attn(b_first, 0).astype(jnp.bfloat16)
        pl.semaphore_wait(barrier_sem, 1)
        s0a, _ = exchange(ctx_ref.at[pl.ds(0, CH)],
                          ctx_rcv_ref.at[pl.ds(0, CH)], 0, p ^ 1)
        ctx_ref[CH:2 * CH, :] = compute_attn(b_first, CH).astype(jnp.bfloat16)
        s0b, _ = exchange(ctx_ref.at[pl.ds(CH, CH)],
                          ctx_rcv_ref.at[pl.ds(CH, CH)], 11, p ^ 1)
        colq = (p ^ 1) * HEAD_BLK
        wo_p = wo_ref[pl.ds(col0, HEAD_BLK), :].astype(jnp.bfloat16)
        wo_q = wo_ref[pl.ds(colq, HEAD_BLK), :].astype(jnp.bfloat16)
        ck = {0: compute_attn(1 - b_first, 0)}
        s0 = {0: s0a, 1: s0b}
        pl.semaphore_wait(yz_sem, 2)

        def klo(ci):
            return pl.multiple_of(keep_lo + ci * CH, 64)

        def fe_start(ci, c0, sem, partner, roff):
            s, _ = exchange(
                acc_ref.at[pl.ds(klo(ci), CH), pl.ds(c0, HALF_C)],
                recv_ref.at[pl.ds(roff + ci * CH, CH), pl.ds(c0, HALF_C)],
                sem, partner,
            )
            return s

        def fe_finish(ci, c0, d, roff):
            d.wait()
            acc_ref[pl.ds(klo(ci), CH), pl.ds(c0, HALF_C)] = (
                acc_ref[pl.ds(klo(ci), CH), pl.ds(c0, HALF_C)]
                + recv_ref[pl.ds(roff + ci * CH, CH), pl.ds(c0, HALF_C)])

        a1, b1, a2, b2, ag = {}, {}, {}, {}, {}
        for ci in range(2):
            s0[ci].wait()
            part = (
                jnp.dot(ck[ci].astype(jnp.bfloat16),
                        wo_p, preferred_element_type=jnp.float32)
                + jnp.dot(ctx_rcv_ref[pl.ds(ci * CH, CH), :], wo_q,
                          preferred_element_type=jnp.float32))
            acc_ref[pl.ds(klo(ci), CH), :] = part.astype(jnp.bfloat16)
            a1[ci] = fe_start(ci, 0, 1 + ci, p ^ 3, 0)
            b1[ci] = fe_start(ci, HALF_C, 3 + ci, p ^ 4, 0)
            if ci == 0:
                ck[1] = compute_attn(1 - b_first, CH)
        for ci in range(2):
            fe_finish(ci, 0, a1[ci], 0)
            a2[ci] = fe_start(ci, 0, 5 + ci, p ^ 4, HALF_R)
            fe_finish(ci, HALF_C, b1[ci], 0)
            b2[ci] = fe_start(ci, HALF_C, 7 + ci, p ^ 3, HALF_R)
        for ci in range(2):
            fe_finish(ci, 0, a2[ci], HALF_R)
            fe_finish(ci, HALF_C, b2[ci], HALF_R)
            slo = pl.multiple_of(send_lo + ci * CH, 64)
            ag[ci] = exchange(
                acc_ref.at[pl.ds(klo(ci), CH)],
                acc_ref.at[pl.ds(klo(ci), CH)],
                9 + ci, p ^ 1,
                recv_dst_slc=acc_ref.at[pl.ds(slo, CH)],
            )
            out_ref[pl.ds(klo(ci), CH), :] = (
                acc_ref[pl.ds(klo(ci), CH), :].astype(jnp.float32))
        for ci in range(2):
            ag_send, ag_recv = ag[ci]
            ag_recv.wait_recv()
            ag_send.wait_send()
            slo = pl.multiple_of(send_lo + ci * CH, 64)
            out_ref[pl.ds(slo, CH), :] = (
                acc_ref[pl.ds(slo, CH), :].astype(jnp.float32))

    out_shape = jax.ShapeDtypeStruct((ROWS, D_MODEL), jnp.float32)
    res = pl.pallas_call(
        body,
        out_shape=out_shape,
        in_specs=[pl.BlockSpec(memory_space=pltpu.VMEM)] * 5,
        out_specs=pl.BlockSpec(memory_space=pltpu.VMEM),
        scratch_shapes=[
            pltpu.VMEM((ROWS, D_MODEL), jnp.bfloat16),
            pltpu.VMEM((2 * HALF_R, D_MODEL), jnp.bfloat16),
            pltpu.VMEM((Sq, HEAD_BLK), jnp.bfloat16),
            pltpu.VMEM((Sq, HEAD_BLK), jnp.bfloat16),
            pltpu.SemaphoreType.DMA((12,)),
            pltpu.SemaphoreType.DMA((12,)),
            pltpu.SemaphoreType.REGULAR,
        ],
        compiler_params=pltpu.CompilerParams(collective_id=0),
    )(x, Wq, K_ext, V_ext, Wo)
    return res.reshape(B, Sq, D_MODEL)
